# baseline (speedup 1.0000x reference)
import functools

import jax
import jax.numpy as jnp
from jax import lax
from jax.experimental import pallas as pl
from jax.experimental.pallas import tpu as pltpu
from jax.experimental.pallas import tpu_sc as plsc

STEPS_PER_DAY = 288
DOW_ROWS = 7
TE_DIM = 64
PAD_DIM = 128
B, T = 4096, 200
FUSED_ROWS = 8 * STEPS_PER_DAY
ROWS = B * T

NUM_CORES = 2
NUM_SUBCORES = 16
NW = NUM_CORES * NUM_SUBCORES
B_PER_W = B // NW
PER_W = ROWS // NW
N_PAIR = B_PER_W // 2
SLICE_A = 104
SLICE_B = T - SLICE_A
STAGE = 6400
UNROLL = 16
LANES = 16


def _fused_table_body(tod_ref, dow_ref, out_ref):
    tod = tod_ref[...].reshape(1, STEPS_PER_DAY, TE_DIM)
    dow = dow_ref[...].reshape(DOW_ROWS, 1, TE_DIM)
    out_ref[0:DOW_ROWS, :, 0:TE_DIM] = tod + dow
    out_ref[DOW_ROWS:8, :, 0:TE_DIM] = jnp.zeros(
        (8 - DOW_ROWS, STEPS_PER_DAY, TE_DIM), jnp.float32
    )
    out_ref[:, :, TE_DIM:PAD_DIM] = jnp.zeros(
        (8, STEPS_PER_DAY, PAD_DIM - TE_DIM), jnp.float32
    )


def _build_fused_table(tod_table, dow_table):
    f3 = pl.pallas_call(
        _fused_table_body,
        out_shape=jax.ShapeDtypeStruct((8, STEPS_PER_DAY, PAD_DIM), jnp.float32),
    )(tod_table, dow_table)
    return f3.reshape(FUSED_ROWS, PAD_DIM)


_MESH = plsc.VectorSubcoreMesh(core_axis_name="c", subcore_axis_name="s")


@functools.partial(
    pl.kernel,
    mesh=_MESH,
    out_type=jax.ShapeDtypeStruct((B, T, PAD_DIM), jnp.float32),
    scratch_types=[
        pltpu.VMEM((STAGE,), jnp.int32),
        pltpu.VMEM((STAGE,), jnp.int32),
        pltpu.VMEM((PER_W,), jnp.int32),
        pltpu.VMEM((T, PAD_DIM), jnp.float32),
        pltpu.VMEM((T, PAD_DIM), jnp.float32),
        pltpu.SemaphoreType.DMA,
        pltpu.SemaphoreType.DMA,
        pltpu.SemaphoreType.DMA,
        pltpu.SemaphoreType.DMA,
        pltpu.VMEM_SHARED((FUSED_ROWS, PAD_DIM), jnp.float32),
    ],
)
def _sc_gather(tod_hbm, dow_hbm, fused_hbm, out_hbm,
               tod_v, dow_v, idx_v, rows0, rows1, sg0, sg1, sw0, sw1, fshared):
    sid = lax.axis_index("s")
    wid = sid * NUM_CORES + lax.axis_index("c")
    base_w = wid * PER_W
    b_w = wid * B_PER_W

    rows_per_sub = FUSED_ROWS // NUM_SUBCORES
    pltpu.sync_copy(
        fused_hbm.at[pl.ds(sid * rows_per_sub, rows_per_sub), :],
        fshared.at[pl.ds(sid * rows_per_sub, rows_per_sub), :],
    )
    plsc.subcore_barrier()

    for sb in range(PER_W // STAGE):
        sbase = sb * STAGE
        pltpu.sync_copy(tod_hbm.at[pl.ds(base_w + sbase, STAGE)], tod_v)
        pltpu.sync_copy(dow_hbm.at[pl.ds(base_w + sbase, STAGE)], dow_v)

        def idx_body(k, carry, sbase=sbase):
            for i in range(UNROLL):
                off = k * (UNROLL * LANES) + i * LANES
                sl = pl.ds(off, LANES)
                idx_v[pl.ds(sbase + off, LANES)] = (
                    tod_v[sl]
                    + STEPS_PER_DAY * jnp.minimum(dow_v[sl], DOW_ROWS - 1)
                )
            return carry

        lax.fori_loop(0, STAGE // (UNROLL * LANES), idx_body, 0)

    def fire_gather(ci, rows, sem):
        pltpu.async_copy(
            fshared.at[idx_v.at[pl.ds(ci * T, T)]],
            rows,
            sem,
        )

    def wait_gather(rows, sem):
        pltpu.make_async_copy(fused_hbm.at[pl.ds(0, T), :], rows, sem).wait()

    def fire_write(ci, rows, sem):
        pltpu.async_copy(rows, out_hbm.at[b_w + ci], sem)

    def wait_write(rows, sem):
        pltpu.make_async_copy(rows, out_hbm.at[0], sem).wait()

    fire_gather(0, rows0, sg0)

    def pair_body(g, carry):
        c0 = 2 * g
        c1 = c0 + 1

        @pl.when(g > 0)
        def _():
            wait_write(rows1, sw1)

        fire_gather(c1, rows1, sg1)
        wait_gather(rows0, sg0)
        fire_write(c0, rows0, sw0)
        wait_gather(rows1, sg1)
        fire_write(c1, rows1, sw1)

        @pl.when(g < N_PAIR - 1)
        def _():
            wait_write(rows0, sw0)
            fire_gather(c0 + 2, rows0, sg0)

        return carry

    lax.fori_loop(0, N_PAIR, pair_body, 0)
    wait_write(rows0, sw0)
    wait_write(rows1, sw1)


def kernel(te, tod_table, dow_table):
    fused = _build_fused_table(tod_table, dow_table)
    tod_ids = te[..., 0].reshape(ROWS)
    dow_ids = te[..., 1].reshape(ROWS)
    return _sc_gather(tod_ids, dow_ids, fused)[..., :TE_DIM]


# speedup vs baseline: 1.0002x; 1.0002x over previous
import functools

import jax
import jax.numpy as jnp
from jax import lax
from jax.experimental import pallas as pl
from jax.experimental.pallas import tpu as pltpu
from jax.experimental.pallas import tpu_sc as plsc

STEPS_PER_DAY = 288
DOW_ROWS = 7
TE_DIM = 64
PAD_DIM = 128
B, T = 4096, 200
FUSED_ROWS = 8 * STEPS_PER_DAY
ROWS = B * T

NUM_CORES = 2
NUM_SUBCORES = 16
NW = NUM_CORES * NUM_SUBCORES
B_PER_W = B // NW
PER_W = ROWS // NW
N_PAIR = B_PER_W // 2
STAGE = 6400
UNROLL = 16
LANES = 16


def _fused_table_body(tod_ref, dow_ref, out_ref):
    tod = tod_ref[...].reshape(1, STEPS_PER_DAY, TE_DIM)
    dow = dow_ref[...].reshape(DOW_ROWS, 1, TE_DIM)
    out_ref[0:DOW_ROWS, :, 0:TE_DIM] = tod + dow
    out_ref[DOW_ROWS:8, :, 0:TE_DIM] = jnp.zeros(
        (8 - DOW_ROWS, STEPS_PER_DAY, TE_DIM), jnp.float32
    )
    out_ref[:, :, TE_DIM:PAD_DIM] = jnp.zeros(
        (8, STEPS_PER_DAY, PAD_DIM - TE_DIM), jnp.float32
    )


def _build_fused_table(tod_table, dow_table):
    f3 = pl.pallas_call(
        _fused_table_body,
        out_shape=jax.ShapeDtypeStruct((8, STEPS_PER_DAY, PAD_DIM), jnp.float32),
    )(tod_table, dow_table)
    return f3.reshape(FUSED_ROWS, PAD_DIM)


_MESH = plsc.VectorSubcoreMesh(core_axis_name="c", subcore_axis_name="s")


@functools.partial(
    pl.kernel,
    mesh=_MESH,
    out_type=jax.ShapeDtypeStruct((B, T, PAD_DIM), jnp.float32),
    scratch_types=[
        pltpu.VMEM((STAGE,), jnp.int32),
        pltpu.VMEM((STAGE,), jnp.int32),
        pltpu.VMEM((PER_W,), jnp.int32),
        pltpu.VMEM((T, PAD_DIM), jnp.float32),
        pltpu.VMEM((T, PAD_DIM), jnp.float32),
        pltpu.SemaphoreType.DMA,
        pltpu.SemaphoreType.DMA,
        pltpu.SemaphoreType.DMA,
        pltpu.SemaphoreType.DMA,
        pltpu.VMEM_SHARED((FUSED_ROWS, PAD_DIM), jnp.float32),
    ],
)
def _sc_gather(tod_hbm, dow_hbm, fused_hbm, out_hbm,
               tod_v, dow_v, idx_v, rows0, rows1, sg0, sg1, sw0, sw1, fshared):
    sid = lax.axis_index("s")
    wid = sid * NUM_CORES + lax.axis_index("c")
    base_w = wid * PER_W
    b_w = wid * B_PER_W

    rows_per_sub = FUSED_ROWS // NUM_SUBCORES
    pltpu.sync_copy(
        fused_hbm.at[pl.ds(sid * rows_per_sub, rows_per_sub), :],
        fshared.at[pl.ds(sid * rows_per_sub, rows_per_sub), :],
    )
    plsc.subcore_barrier()

    for sb in range(PER_W // STAGE):
        sbase = sb * STAGE
        pltpu.sync_copy(tod_hbm.at[pl.ds(base_w + sbase, STAGE)], tod_v)
        pltpu.sync_copy(dow_hbm.at[pl.ds(base_w + sbase, STAGE)], dow_v)

        def idx_body(k, carry, sbase=sbase):
            for i in range(UNROLL):
                off = k * (UNROLL * LANES) + i * LANES
                sl = pl.ds(off, LANES)
                idx_v[pl.ds(sbase + off, LANES)] = (
                    tod_v[sl]
                    + STEPS_PER_DAY * jnp.minimum(dow_v[sl], DOW_ROWS - 1)
                )
            return carry

        lax.fori_loop(0, STAGE // (UNROLL * LANES), idx_body, 0)

    def fire_gather(ci, rows, sem):
        pltpu.async_copy(
            fshared.at[idx_v.at[pl.ds(ci * T, T)]],
            rows,
            sem,
        )

    def wait_gather(rows, sem):
        pltpu.make_async_copy(fused_hbm.at[pl.ds(0, T), :], rows, sem).wait()

    def fire_write(ci, rows, sem):
        pltpu.async_copy(rows, out_hbm.at[b_w + ci], sem)

    def wait_write(rows, sem):
        pltpu.make_async_copy(rows, out_hbm.at[0], sem).wait()

    fire_gather(0, rows0, sg0)

    def pair_body(g, carry):
        c0 = 2 * g
        c1 = c0 + 1

        @pl.when(g > 0)
        def _():
            wait_write(rows1, sw1)

        fire_gather(c1, rows1, sg1)
        wait_gather(rows0, sg0)
        fire_write(c0, rows0, sw0)
        wait_gather(rows1, sg1)
        fire_write(c1, rows1, sw1)

        @pl.when(g < N_PAIR - 1)
        def _():
            wait_write(rows0, sw0)
            fire_gather(c0 + 2, rows0, sg0)

        return carry

    lax.fori_loop(0, N_PAIR, pair_body, 0)
    wait_write(rows0, sw0)
    wait_write(rows1, sw1)


def kernel(te, tod_table, dow_table):
    fused = _build_fused_table(tod_table, dow_table)
    tod_ids = te[..., 0].reshape(ROWS)
    dow_ids = te[..., 1].reshape(ROWS)
    return _sc_gather(tod_ids, dow_ids, fused)[..., :TE_DIM]


# speedup vs baseline: 1.0138x; 1.0137x over previous
import functools

import jax
import jax.numpy as jnp
from jax import lax
from jax.experimental import pallas as pl
from jax.experimental.pallas import tpu as pltpu
from jax.experimental.pallas import tpu_sc as plsc

STEPS_PER_DAY = 288
DOW_ROWS = 7
TE_DIM = 64
PAD_DIM = 128
B, T = 4096, 200
FUSED_ROWS = 8 * STEPS_PER_DAY
ROWS = B * T

NUM_CORES = 2
NUM_SUBCORES = 16
NW = NUM_CORES * NUM_SUBCORES
B_PER_W = B // NW
PER_W = ROWS // NW
N_PAIR = B_PER_W // 2
STAGE = 6400
UNROLL = 16
LANES = 16


def _fused_table_body(tod_ref, dow_ref, out_ref):
    tod = tod_ref[...].reshape(1, STEPS_PER_DAY, TE_DIM)
    dow = dow_ref[...].reshape(DOW_ROWS, 1, TE_DIM)
    out_ref[0:DOW_ROWS, :, 0:TE_DIM] = tod + dow
    out_ref[DOW_ROWS:8, :, 0:TE_DIM] = jnp.zeros(
        (8 - DOW_ROWS, STEPS_PER_DAY, TE_DIM), jnp.float32
    )
    out_ref[:, :, TE_DIM:PAD_DIM] = jnp.zeros(
        (8, STEPS_PER_DAY, PAD_DIM - TE_DIM), jnp.float32
    )


def _build_fused_table(tod_table, dow_table):
    f3 = pl.pallas_call(
        _fused_table_body,
        out_shape=jax.ShapeDtypeStruct((8, STEPS_PER_DAY, PAD_DIM), jnp.float32),
    )(tod_table, dow_table)
    return f3.reshape(FUSED_ROWS, PAD_DIM)


_MESH = plsc.VectorSubcoreMesh(core_axis_name="c", subcore_axis_name="s")


@functools.partial(
    pl.kernel,
    mesh=_MESH,
    out_type=jax.ShapeDtypeStruct((B, T, PAD_DIM), jnp.float32),
    scratch_types=[
        pltpu.VMEM((STAGE,), jnp.int32),
        pltpu.VMEM((STAGE,), jnp.int32),
        pltpu.VMEM((PER_W,), jnp.int32),
        pltpu.VMEM((T, PAD_DIM), jnp.float32),
        pltpu.VMEM((T, PAD_DIM), jnp.float32),
        pltpu.SemaphoreType.DMA,
        pltpu.SemaphoreType.DMA,
        pltpu.SemaphoreType.DMA,
        pltpu.SemaphoreType.DMA,
        pltpu.VMEM_SHARED((FUSED_ROWS, PAD_DIM), jnp.float32),
    ],
)
def _sc_gather(tod_hbm, dow_hbm, fused_hbm, out_hbm,
               tod_v, dow_v, idx_v, rows0, rows1, sg0, sg1, sw0, sw1, fshared):
    sid = lax.axis_index("s")
    wid = sid * NUM_CORES + lax.axis_index("c")
    base_w = wid * PER_W
    b_w = wid * B_PER_W

    rows_per_sub = FUSED_ROWS // NUM_SUBCORES
    pltpu.sync_copy(
        fused_hbm.at[pl.ds(sid * rows_per_sub, rows_per_sub), :],
        fshared.at[pl.ds(sid * rows_per_sub, rows_per_sub), :],
    )
    plsc.subcore_barrier()

    def compute_idx_block(sb):
        sbase = sb * STAGE
        pltpu.sync_copy(tod_hbm.at[pl.ds(base_w + sbase, STAGE)], tod_v)
        pltpu.sync_copy(dow_hbm.at[pl.ds(base_w + sbase, STAGE)], dow_v)

        def idx_body(k, carry):
            for i in range(UNROLL):
                off = k * (UNROLL * LANES) + i * LANES
                sl = pl.ds(off, LANES)
                idx_v[pl.ds(sbase + off, LANES)] = (
                    tod_v[sl]
                    + STEPS_PER_DAY * jnp.minimum(dow_v[sl], DOW_ROWS - 1)
                )
            return carry

        lax.fori_loop(0, STAGE // (UNROLL * LANES), idx_body, 0)

    compute_idx_block(0)

    def fire_gather(ci, rows, sem):
        pltpu.async_copy(
            fshared.at[idx_v.at[pl.ds(ci * T, T)]],
            rows,
            sem,
        )

    def wait_gather(rows, sem):
        pltpu.make_async_copy(fused_hbm.at[pl.ds(0, T), :], rows, sem).wait()

    def fire_write(ci, rows, sem):
        pltpu.async_copy(rows, out_hbm.at[b_w + ci], sem)

    def wait_write(rows, sem):
        pltpu.make_async_copy(rows, out_hbm.at[0], sem).wait()

    fire_gather(0, rows0, sg0)

    def pair_body(g, carry):
        c0 = 2 * g
        c1 = c0 + 1

        @pl.when(g > 0)
        def _():
            wait_write(rows1, sw1)

        fire_gather(c1, rows1, sg1)
        for trigger_g, blk in ((10, 1), (26, 2), (42, 3)):
            @pl.when(g == trigger_g)
            def _(blk=blk):
                compute_idx_block(blk)

        wait_gather(rows0, sg0)
        fire_write(c0, rows0, sw0)
        wait_gather(rows1, sg1)
        fire_write(c1, rows1, sw1)

        @pl.when(g < N_PAIR - 1)
        def _():
            wait_write(rows0, sw0)
            fire_gather(c0 + 2, rows0, sg0)

        return carry

    lax.fori_loop(0, N_PAIR, pair_body, 0)
    wait_write(rows0, sw0)
    wait_write(rows1, sw1)


def kernel(te, tod_table, dow_table):
    fused = _build_fused_table(tod_table, dow_table)
    tod_ids = te[..., 0].reshape(ROWS)
    dow_ids = te[..., 1].reshape(ROWS)
    return _sc_gather(tod_ids, dow_ids, fused)[..., :TE_DIM]
